# initial kernel scaffold (unmeasured)
import jax
import jax.numpy as jnp
from jax import lax
from jax.experimental import pallas as pl
from jax.experimental.pallas import tpu as pltpu


def kernel(partial, resid, gamma):
    _, M, D = partial.shape

    def body(partial_ref, resid_ref, gamma_ref, out_ref, comm_ref,
             send_sem, recv_sem):
        my_x = lax.axis_index("x")
        my_y = lax.axis_index("y")
        my_z = lax.axis_index("z")
        peer = (1 - my_x, my_y, my_z)

        barrier_sem = pltpu.get_barrier_semaphore()
        pl.semaphore_signal(
            barrier_sem, inc=1, device_id=peer,
            device_id_type=pl.DeviceIdType.MESH,
        )
        pl.semaphore_wait(barrier_sem, 1)

        comm_ref[0] = partial_ref[0].astype(jnp.bfloat16)
        rdma = pltpu.make_async_remote_copy(
            src_ref=comm_ref.at[0],
            dst_ref=comm_ref.at[1],
            send_sem=send_sem,
            recv_sem=recv_sem,
            device_id=peer,
            device_id_type=pl.DeviceIdType.MESH,
        )
        rdma.start()
        rdma.wait()

        y = (partial_ref[0]
             + comm_ref[1].astype(jnp.float32)
             + resid_ref[...])
        rms = jnp.sqrt(jnp.mean(y * y, axis=-1, keepdims=True) + 1e-6)
        out_ref[...] = y / rms * gamma_ref[...][None, :]

    return pl.pallas_call(
        body,
        out_shape=jax.ShapeDtypeStruct((M, D), jnp.float32),
        in_specs=[pl.BlockSpec(memory_space=pltpu.VMEM)] * 3,
        out_specs=pl.BlockSpec(memory_space=pltpu.VMEM),
        scratch_shapes=[
            pltpu.VMEM((2, M, D), jnp.bfloat16),
            pltpu.SemaphoreType.DMA,
            pltpu.SemaphoreType.DMA,
        ],
        compiler_params=pltpu.CompilerParams(collective_id=0),
    )(partial, resid, gamma)


# baseline (device time: 130825 ns/iter reference)
import jax
import jax.numpy as jnp
from jax import lax
from jax.experimental import pallas as pl
from jax.experimental.pallas import tpu as pltpu

CHUNK = 256


def kernel(partial, resid, gamma):
    _, M, D = partial.shape
    n_chunks = M // CHUNK

    def body(partial_ref, resid_ref, gamma_ref, out_ref, comm_ref,
             pchunk, rchunk, send_sem, recv_sem, dma_sem):
        my_x = lax.axis_index("x")
        my_y = lax.axis_index("y")
        my_z = lax.axis_index("z")
        peer = (1 - my_x, my_y, my_z)

        barrier_sem = pltpu.get_barrier_semaphore()
        pl.semaphore_signal(
            barrier_sem, inc=1, device_id=peer,
            device_id_type=pl.DeviceIdType.MESH,
        )
        pl.semaphore_wait(barrier_sem, 1)

        for i in range(n_chunks):
            cp = pltpu.make_async_copy(
                partial_ref.at[0, pl.ds(i * CHUNK, CHUNK)], pchunk, dma_sem)
            cp.start()
            cp.wait()
            comm_ref[0, pl.ds(i * CHUNK, CHUNK)] = (
                pchunk[...].astype(jnp.bfloat16))

        rdma = pltpu.make_async_remote_copy(
            src_ref=comm_ref.at[0],
            dst_ref=comm_ref.at[1],
            send_sem=send_sem,
            recv_sem=recv_sem,
            device_id=peer,
            device_id_type=pl.DeviceIdType.MESH,
        )
        rdma.start()
        rdma.wait()

        for i in range(n_chunks):
            cp = pltpu.make_async_copy(
                resid_ref.at[pl.ds(i * CHUNK, CHUNK)], rchunk, dma_sem)
            cp.start()
            cp.wait()
            rows = pl.ds(i * CHUNK, CHUNK)
            y = (comm_ref[0, rows].astype(jnp.float32)
                 + comm_ref[1, rows].astype(jnp.float32)
                 + rchunk[...])
            rms = jnp.sqrt(jnp.mean(y * y, axis=-1, keepdims=True) + 1e-6)
            out_ref[rows, :] = y / rms * gamma_ref[...][None, :]

    return pl.pallas_call(
        body,
        out_shape=jax.ShapeDtypeStruct((M, D), jnp.float32),
        in_specs=[
            pl.BlockSpec(memory_space=pltpu.MemorySpace.HBM),
            pl.BlockSpec(memory_space=pltpu.MemorySpace.HBM),
            pl.BlockSpec(memory_space=pltpu.VMEM),
        ],
        out_specs=pl.BlockSpec(memory_space=pltpu.VMEM),
        scratch_shapes=[
            pltpu.VMEM((2, M, D), jnp.bfloat16),
            pltpu.VMEM((CHUNK, D), jnp.float32),
            pltpu.VMEM((CHUNK, D), jnp.float32),
            pltpu.SemaphoreType.DMA,
            pltpu.SemaphoreType.DMA,
            pltpu.SemaphoreType.DMA,
        ],
        compiler_params=pltpu.CompilerParams(collective_id=0),
    )(partial, resid, gamma)


# device time: 74570 ns/iter; 1.7544x vs baseline; 1.7544x over previous
import jax
import jax.numpy as jnp
from jax import lax
from jax.experimental import pallas as pl
from jax.experimental.pallas import tpu as pltpu

N_LINE = 4
R = 128


def kernel(partial, resid, gamma):
    _, M, D = partial.shape
    HC = D // 2

    def body(partial_ref, resid_ref, gamma_ref, out_ref,
             gathA, gathB, pstage, rstage, psend, precv,
             send_x, recv_x, dma_p, dma_r, ag_send, ag_recv):
        my_x = lax.axis_index("x")
        my_y = lax.axis_index("y")
        my_z = lax.axis_index("z")
        xpeer = (1 - my_x, my_y, my_z)

        barrier_sem = pltpu.get_barrier_semaphore()
        pl.semaphore_signal(barrier_sem, inc=1, device_id=xpeer,
                            device_id_type=pl.DeviceIdType.MESH)

        @pl.when(my_y > 0)
        def _():
            pl.semaphore_signal(barrier_sem, inc=1,
                                device_id=(my_x, my_y - 1, my_z),
                                device_id_type=pl.DeviceIdType.MESH)

        @pl.when(my_y < N_LINE - 1)
        def _():
            pl.semaphore_signal(barrier_sem, inc=1,
                                device_id=(my_x, my_y + 1, my_z),
                                device_id_type=pl.DeviceIdType.MESH)

        @pl.when(my_z > 0)
        def _():
            pl.semaphore_signal(barrier_sem, inc=1,
                                device_id=(my_x, my_y, my_z - 1),
                                device_id_type=pl.DeviceIdType.MESH)

        @pl.when(my_z < N_LINE - 1)
        def _():
            pl.semaphore_signal(barrier_sem, inc=1,
                                device_id=(my_x, my_y, my_z + 1),
                                device_id_type=pl.DeviceIdType.MESH)

        n_nbrs = (1
                  + (my_y > 0).astype(jnp.int32)
                  + (my_y < N_LINE - 1).astype(jnp.int32)
                  + (my_z > 0).astype(jnp.int32)
                  + (my_z < N_LINE - 1).astype(jnp.int32))
        pl.semaphore_wait(barrier_sem, n_nbrs)

        c_me = N_LINE * my_y + my_z
        row0 = c_me * R
        cp = pltpu.make_async_copy(
            partial_ref.at[0, pl.ds(row0, R)], pstage, dma_p)
        cp.start()
        cr = pltpu.make_async_copy(
            resid_ref.at[pl.ds(row0, R)], rstage, dma_r)
        cr.start()
        cp.wait()
        psend[...] = pstage[...].astype(jnp.bfloat16)
        rx = pltpu.make_async_remote_copy(
            src_ref=psend, dst_ref=precv, send_sem=send_x, recv_sem=recv_x,
            device_id=xpeer, device_id_type=pl.DeviceIdType.MESH)
        rx.start()
        rx.wait()
        cr.wait()

        y = (psend[...].astype(jnp.float32)
             + precv[...].astype(jnp.float32)
             + rstage[...])
        rms = jnp.sqrt(jnp.mean(y * y, axis=-1, keepdims=True) + 1e-6)
        o = y / rms * gamma_ref[...][None, :]
        gathA[my_y, my_z] = o[:, :HC].astype(jnp.bfloat16)
        gathB[my_z, my_y] = o[:, HC:].astype(jnp.bfloat16)

        def z_peer(d):
            return (my_x, my_y, my_z + d)

        def y_peer(d):
            return (my_x, my_y + d, my_z)

        def unit_A_z(i):
            return gathA.at[my_y, i]

        def unit_B_y(i):
            return gathB.at[my_z, i]

        def unit_A_y(i):
            return gathA.at[i]

        def unit_B_z(i):
            return gathB.at[i]

        def step_send(phase, s, pos, unit, peer_fn):
            @pl.when((pos >= s) & (pos < N_LINE - 1))
            def _():
                r = pltpu.make_async_remote_copy(
                    src_ref=unit(pos - s), dst_ref=unit(pos - s),
                    send_sem=ag_send.at[phase, s, 0],
                    recv_sem=ag_recv.at[phase, s, 0],
                    device_id=peer_fn(1),
                    device_id_type=pl.DeviceIdType.MESH)
                r.start()

            @pl.when((pos > 0) & (pos + s <= N_LINE - 1))
            def _():
                r = pltpu.make_async_remote_copy(
                    src_ref=unit(pos + s), dst_ref=unit(pos + s),
                    send_sem=ag_send.at[phase, s, 1],
                    recv_sem=ag_recv.at[phase, s, 1],
                    device_id=peer_fn(-1),
                    device_id_type=pl.DeviceIdType.MESH)
                r.start()

        def step_wait(phase, s, pos, unit, peer_fn):
            @pl.when(pos >= s + 1)
            def _():
                r = pltpu.make_async_remote_copy(
                    src_ref=unit(pos - 1 - s), dst_ref=unit(pos - 1 - s),
                    send_sem=ag_send.at[phase, s, 0],
                    recv_sem=ag_recv.at[phase, s, 0],
                    device_id=peer_fn(-1),
                    device_id_type=pl.DeviceIdType.MESH)
                r.wait_recv()

            @pl.when(pos + 1 + s <= N_LINE - 1)
            def _():
                r = pltpu.make_async_remote_copy(
                    src_ref=unit(pos + 1 + s), dst_ref=unit(pos + 1 + s),
                    send_sem=ag_send.at[phase, s, 1],
                    recv_sem=ag_recv.at[phase, s, 1],
                    device_id=peer_fn(1),
                    device_id_type=pl.DeviceIdType.MESH)
                r.wait_recv()

            @pl.when((pos >= s) & (pos < N_LINE - 1))
            def _():
                r = pltpu.make_async_remote_copy(
                    src_ref=unit(pos - s), dst_ref=unit(pos - s),
                    send_sem=ag_send.at[phase, s, 0],
                    recv_sem=ag_recv.at[phase, s, 0],
                    device_id=peer_fn(1),
                    device_id_type=pl.DeviceIdType.MESH)
                r.wait_send()

            @pl.when((pos > 0) & (pos + s <= N_LINE - 1))
            def _():
                r = pltpu.make_async_remote_copy(
                    src_ref=unit(pos + s), dst_ref=unit(pos + s),
                    send_sem=ag_send.at[phase, s, 1],
                    recv_sem=ag_recv.at[phase, s, 1],
                    device_id=peer_fn(-1),
                    device_id_type=pl.DeviceIdType.MESH)
                r.wait_send()

        for s in range(N_LINE - 1):
            step_send(0, s, my_z, unit_A_z, z_peer)
            step_send(1, s, my_y, unit_B_y, y_peer)
            step_wait(0, s, my_z, unit_A_z, z_peer)
            step_wait(1, s, my_y, unit_B_y, y_peer)

        for s in range(N_LINE - 1):
            step_send(2, s, my_y, unit_A_y, y_peer)
            step_send(3, s, my_z, unit_B_z, z_peer)
            step_wait(2, s, my_y, unit_A_y, y_peer)
            step_wait(3, s, my_z, unit_B_z, z_peer)

        for yy in range(N_LINE):
            for zz in range(N_LINE):
                c = N_LINE * yy + zz
                rows = pl.ds(c * R, R)
                out_ref[rows, 0:HC] = gathA[yy, zz].astype(jnp.float32)
                out_ref[rows, HC:D] = gathB[zz, yy].astype(jnp.float32)

    return pl.pallas_call(
        body,
        out_shape=jax.ShapeDtypeStruct((M, D), jnp.float32),
        in_specs=[
            pl.BlockSpec(memory_space=pltpu.MemorySpace.HBM),
            pl.BlockSpec(memory_space=pltpu.MemorySpace.HBM),
            pl.BlockSpec(memory_space=pltpu.MemorySpace.VMEM),
        ],
        out_specs=pl.BlockSpec(memory_space=pltpu.MemorySpace.VMEM),
        scratch_shapes=[
            pltpu.VMEM((N_LINE, N_LINE, R, HC), jnp.bfloat16),
            pltpu.VMEM((N_LINE, N_LINE, R, HC), jnp.bfloat16),
            pltpu.VMEM((R, D), jnp.float32),
            pltpu.VMEM((R, D), jnp.float32),
            pltpu.VMEM((R, D), jnp.bfloat16),
            pltpu.VMEM((R, D), jnp.bfloat16),
            pltpu.SemaphoreType.DMA,
            pltpu.SemaphoreType.DMA,
            pltpu.SemaphoreType.DMA,
            pltpu.SemaphoreType.DMA,
            pltpu.SemaphoreType.DMA((4, N_LINE - 1, 2)),
            pltpu.SemaphoreType.DMA((4, N_LINE - 1, 2)),
        ],
        compiler_params=pltpu.CompilerParams(collective_id=0),
    )(partial, resid, gamma)
